# baseline (speedup 1.0000x reference)
import functools

import jax
import jax.numpy as jnp
from jax import lax
from jax.experimental import pallas as pl
from jax.experimental.pallas import tpu as pltpu
from jax.experimental.pallas import tpu_sc as plsc

B, C, N, K = 2, 128, 10000, 32
NC, NS, L = 2, 16, 16
SEG = N // NS
G = 4
SEG0 = 624
NT1 = 12
CHG = 78
CH = CHG * G
SROWS = 125
EROWS = B * N * K // 128
TBLK = 2000

_ILV = plsc.PackFormat.INTERLEAVED


def _sc_gather_mean(xt, edges1):
    @functools.partial(
        pl.kernel,
        mesh=plsc.VectorSubcoreMesh(core_axis_name="c", subcore_axis_name="s"),
        out_type=jax.ShapeDtypeStruct((B, N, C), jnp.float32),
        compiler_params=pltpu.CompilerParams(use_tc_tiling_on_sc=False,
                                             needs_layout_passes=False),
        scratch_types=[
            pltpu.VMEM_SHARED((N, C), jnp.bfloat16),
            pltpu.VMEM((157, 128), jnp.int32),
            pltpu.VMEM((CH, C), jnp.float32),
            pltpu.VMEM((G * K, C), jnp.bfloat16),
            pltpu.VMEM((G * K, C), jnp.bfloat16),
            pltpu.SemaphoreType.DMA,
            pltpu.SemaphoreType.DMA,
        ],
    )
    def k(xt_hbm, edges_hbm, out_hbm, x_sh, idx_v, agg_a, r0, r1, sem0, sem1):
        c = lax.axis_index("c")
        s = lax.axis_index("s")
        extra = jnp.maximum(s - NT1, 0)
        base = s * SEG0 + G * extra

        def stage_in(i, sem):
            row = s * SEG + i * SROWS
            return pltpu.make_async_copy(xt_hbm.at[c, pl.ds(row, SROWS)],
                                         agg_a.at[pl.ds(0, SROWS)], sem)

        def stage_chunk(i, carry):
            stage_in(i, sem0).wait()
            lax.fori_loop(0, SROWS, conv_row, i)
            @pl.when(i + 1 < SEG // SROWS)
            def _():
                stage_in(i + 1, sem0).start()
            pltpu.sync_copy(r0.at[pl.ds(0, SROWS)],
                            x_sh.at[pl.ds(s * SEG + i * SROWS, SROWS)])
            return carry

        def conv_row(rr, i):
            for ccb in range(C // 32):
                a = agg_a[rr, pl.ds(ccb * 32, L)]
                bq = agg_a[rr, pl.ds(ccb * 32 + L, L)]
                r0[rr, pl.ds(ccb * 32, 32)] = plsc.pack(a, bq, format=_ILV)
            return i

        stage_in(0, sem0).start()
        lax.fori_loop(0, SEG // SROWS, stage_chunk, 0)
        erow = (c * N + base) // G
        pltpu.sync_copy(edges_hbm.at[pl.ds(erow, 157)], idx_v)
        plsc.subcore_barrier()

        def fire(gg, rbuf, sem):
            src = x_sh.at[idx_v.at[gg]]
            pltpu.make_async_copy(src, rbuf, sem).start()

        def drain(gg, rbuf, sem):
            src = x_sh.at[idx_v.at[gg]]
            pltpu.make_async_copy(src, rbuf, sem).wait()

        def reduce_node(rbuf, row0):
            outs = []
            for ccb in range(C // 32):
                sl = pl.ds(ccb * 32, 32)
                parts = []
                for p8 in range(K // 8):
                    r = row0 + p8 * 8
                    a0 = rbuf[r + 0, sl] + rbuf[r + 1, sl]
                    a1 = rbuf[r + 2, sl] + rbuf[r + 3, sl]
                    a2 = rbuf[r + 4, sl] + rbuf[r + 5, sl]
                    a3 = rbuf[r + 6, sl] + rbuf[r + 7, sl]
                    parts.append((a0 + a1) + (a2 + a3))
                t = ((parts[0] + parts[1]) + (parts[2] + parts[3])) * (1.0 / K)
                outs.append(plsc.unpack(t, format=_ILV,
                                        preferred_element_type=jnp.float32))
            return outs

        def store_node(aggbuf, outs, out_row):
            for ccb, (lo, hi) in enumerate(outs):
                aggbuf[out_row, pl.ds(ccb * 32, L)] = lo
                aggbuf[out_row, pl.ds(ccb * 32 + L, L)] = hi

        def reduce_group(rbuf, g, aggbuf):
            for t in range(G):
                store_node(aggbuf, reduce_node(rbuf, t * K), g * G + t)

        def chunk_body(q, carry):
            g0 = q * CHG
            fire(g0, r0, sem0)

            def body(i, cc2):
                fire(g0 + 2 * i + 1, r1, sem1)
                drain(g0 + 2 * i, r0, sem0)
                reduce_group(r0, 2 * i, agg_a)

                @pl.when(i < CHG // 2 - 1)
                def _():
                    fire(g0 + 2 * i + 2, r0, sem0)

                drain(g0 + 2 * i + 1, r1, sem1)
                reduce_group(r1, 2 * i + 1, agg_a)
                return cc2

            lax.fori_loop(0, CHG // 2, body, 0)
            pltpu.sync_copy(agg_a, out_hbm.at[c, pl.ds(base + g0 * G, CH)])
            return carry

        lax.fori_loop(0, 2, chunk_body, 0)

        @pl.when(s >= NT1)
        def _extra_group():
            fire(2 * CHG, r0, sem0)
            drain(2 * CHG, r0, sem0)
            reduce_group(r0, 0, agg_a)
            pltpu.sync_copy(agg_a.at[pl.ds(0, G)],
                            out_hbm.at[c, pl.ds(base + 2 * CHG * G, G)])

    return k(xt, edges1)


def _tc_conv_bn_act(agg, W2, gamma2, beta2):
    def body(agg_ref, w_ref, g_ref, b_ref, out_ref, s_acc, m_acc, wp_ref, bp_ref):
        p = pl.program_id(0)
        b = pl.program_id(1)

        @pl.when(p == 0)
        def _phase_stats():
            blk = agg_ref[0].astype(jnp.bfloat16)
            contrib = lax.dot_general(blk, blk, (((0,), (0,)), ((), ())),
                                      preferred_element_type=jnp.float32)
            ones = jnp.ones((N, 1), jnp.bfloat16)
            mcon = lax.dot_general(blk, ones, (((0,), (0,)), ((), ())),
                                   preferred_element_type=jnp.float32)

            @pl.when(b == 0)
            def _init():
                s_acc[...] = contrib
                m_acc[...] = mcon

            @pl.when(b != 0)
            def _accum():
                s_acc[...] = s_acc[...] + contrib
                m_acc[...] = m_acc[...] + mcon

        @pl.when(p == 1)
        def _phase_emit():
            @pl.when(b == 0)
            def _fold_bn():
                cnt = float(B * N)
                w = w_ref[...]
                mean = lax.dot_general(w, m_acc[...], (((1,), (0,)), ((), ())),
                                       preferred_element_type=jnp.float32) / cnt
                ws = lax.dot_general(w, s_acc[...], (((1,), (0,)), ((), ())),
                                     preferred_element_type=jnp.float32)
                ey2 = jnp.sum(ws * w, axis=1, keepdims=True) / cnt
                var = ey2 - mean * mean
                scale = g_ref[...] * lax.rsqrt(var + 1e-5)
                wp_ref[...] = (w * scale).astype(jnp.bfloat16)
                bp_ref[...] = b_ref[...] - mean * scale

            for jj in range(N // TBLK):
                blkj = agg_ref[0, pl.ds(jj * TBLK, TBLK), :].astype(jnp.bfloat16)
                y = lax.dot_general(wp_ref[...], blkj, (((1,), (1,)), ((), ())),
                                    preferred_element_type=jnp.float32)
                y = y + bp_ref[...]
                out_ref[0, :, pl.ds(jj * TBLK, TBLK)] = jnp.where(y >= 0, y, 0.2 * y)

    return pl.pallas_call(
        body,
        grid=(2, B),
        in_specs=[
            pl.BlockSpec((1, N, C), lambda p, b: (b, 0, 0)),
            pl.BlockSpec((C, C), lambda p, b: (0, 0)),
            pl.BlockSpec((C, 1), lambda p, b: (0, 0)),
            pl.BlockSpec((C, 1), lambda p, b: (0, 0)),
        ],
        out_specs=pl.BlockSpec((1, C, N), lambda p, b: (b, 0, 0)),
        out_shape=jax.ShapeDtypeStruct((B, C, N), jnp.float32),
        scratch_shapes=[
            pltpu.VMEM((C, C), jnp.float32),
            pltpu.VMEM((C, 1), jnp.float32),
            pltpu.VMEM((C, C), jnp.bfloat16),
            pltpu.VMEM((C, 1), jnp.float32),
        ],
    )(agg, W2, gamma2, beta2)


def kernel(x, edges, W, gamma, beta):
    xt = jnp.transpose(x, (0, 2, 1))
    edges1 = edges.reshape(EROWS, 128)
    agg = _sc_gather_mean(xt, edges1)
    return _tc_conv_bn_act(agg, W, gamma.reshape(C, 1), beta.reshape(C, 1))
